# baseline (speedup 1.0000x reference)
import math

import jax
import jax.numpy as jnp
from jax.experimental import pallas as pl
from jax.experimental.pallas import tpu as pltpu

N = 50000
E = 800000
HIDDEN = 64
NUM_BASIS = 8
NUM_ATOMS = 100
CUTOFF = 5.0
T = 4096
LUTROWS = T + 8
NBLK = 2000
NGRID = N // NBLK


def _silu(x):
    return x * jax.nn.sigmoid(x)


def _lut_body(centers_ref, widths_ref, w1t_ref, b1_ref, w2t_ref, b2_ref, out_ref):
    t = jax.lax.broadcasted_iota(jnp.int32, (LUTROWS, 1), 0).astype(jnp.float32)
    ln = t * (CUTOFF / T)
    cut = 0.5 * (jnp.cos(ln * (math.pi / CUTOFF)) + 1.0)
    cut = cut * (ln < CUTOFF).astype(jnp.float32)
    diff = ln - centers_ref[...]
    wid = jnp.clip(widths_ref[...], 0.1, None)
    basis = jnp.exp(-0.5 * (diff / wid) ** 2) * cut
    h1 = jnp.dot(basis, w1t_ref[0], preferred_element_type=jnp.float32) + b1_ref[0]
    h1 = _silu(h1)
    w = jnp.dot(h1, w2t_ref[0], preferred_element_type=jnp.float32) + b2_ref[0]
    out_ref[0] = w


def _build_luts(centers, widths, w1t, b1, w2t, b2):
    return pl.pallas_call(
        _lut_body,
        grid=(3,),
        in_specs=[
            pl.BlockSpec((1, NUM_BASIS), lambda l: (0, 0)),
            pl.BlockSpec((1, NUM_BASIS), lambda l: (0, 0)),
            pl.BlockSpec((1, NUM_BASIS, HIDDEN), lambda l: (l, 0, 0)),
            pl.BlockSpec((1, 1, HIDDEN), lambda l: (l, 0, 0)),
            pl.BlockSpec((1, HIDDEN, HIDDEN), lambda l: (l, 0, 0)),
            pl.BlockSpec((1, 1, HIDDEN), lambda l: (l, 0, 0)),
        ],
        out_specs=pl.BlockSpec((1, LUTROWS, HIDDEN), lambda l: (l, 0, 0)),
        out_shape=jax.ShapeDtypeStruct((3, LUTROWS, HIDDEN), jnp.float32),
    )(centers.reshape(1, NUM_BASIS), widths.reshape(1, NUM_BASIS),
      w1t, b1, w2t, b2)


def _embed_body(an_ref, emb_ref, out_ref):
    an = an_ref[...]
    ids = jax.lax.broadcasted_iota(jnp.int32, (NBLK, NUM_ATOMS), 1)
    onehot = (an == ids).astype(jnp.float32)
    out_ref[...] = jnp.dot(onehot, emb_ref[...], preferred_element_type=jnp.float32)


def _embed(atomic_numbers, node_emb):
    return pl.pallas_call(
        _embed_body,
        grid=(NGRID,),
        in_specs=[
            pl.BlockSpec((NBLK, 1), lambda i: (i, 0)),
            pl.BlockSpec((NUM_ATOMS, HIDDEN), lambda i: (0, 0)),
        ],
        out_specs=pl.BlockSpec((NBLK, HIDDEN), lambda i: (i, 0)),
        out_shape=jax.ShapeDtypeStruct((N, HIDDEN), jnp.float32),
    )(atomic_numbers.reshape(N, 1), node_emb)


def _node_body(f_ref, a_ref, siwt_ref, sib_ref, cpwt_ref, cpb_ref,
               u1t_ref, ub1_ref, u2t_ref, ub2_ref, lng_ref, lnb_ref, out_ref):
    f = f_ref[...]
    a = a_ref[...]
    so = jnp.dot(f, siwt_ref[...], preferred_element_type=jnp.float32) + sib_ref[...]
    cpwt = cpwt_ref[...]
    conv = (jnp.dot(so, cpwt[:HIDDEN, :], preferred_element_type=jnp.float32)
            + jnp.dot(a, cpwt[HIDDEN:, :], preferred_element_type=jnp.float32)
            + cpb_ref[...])
    u = _silu(jnp.dot(conv, u1t_ref[...], preferred_element_type=jnp.float32) + ub1_ref[...])
    u = jnp.dot(u, u2t_ref[...], preferred_element_type=jnp.float32) + ub2_ref[...]
    h = f + u
    mu = jnp.mean(h, axis=-1, keepdims=True)
    var = jnp.mean((h - mu) ** 2, axis=-1, keepdims=True)
    out_ref[...] = (h - mu) * jax.lax.rsqrt(var + 1e-5) * lng_ref[...] + lnb_ref[...]


def _node_update(feats, agg, p):
    full = lambda shp: pl.BlockSpec(shp, lambda i: tuple(0 for _ in shp))
    return pl.pallas_call(
        _node_body,
        grid=(NGRID,),
        in_specs=[
            pl.BlockSpec((NBLK, HIDDEN), lambda i: (i, 0)),
            pl.BlockSpec((NBLK, HIDDEN), lambda i: (i, 0)),
            full((HIDDEN, HIDDEN)), full((1, HIDDEN)),
            full((2 * HIDDEN, HIDDEN)), full((1, HIDDEN)),
            full((HIDDEN, 2 * HIDDEN)), full((1, 2 * HIDDEN)),
            full((2 * HIDDEN, HIDDEN)), full((1, HIDDEN)),
            full((1, HIDDEN)), full((1, HIDDEN)),
        ],
        out_specs=pl.BlockSpec((NBLK, HIDDEN), lambda i: (i, 0)),
        out_shape=jax.ShapeDtypeStruct((N, HIDDEN), jnp.float32),
    )(feats, agg,
      p['si_W'].T, p['si_b'].reshape(1, HIDDEN),
      p['cp_W'].T, p['cp_b'].reshape(1, HIDDEN),
      p['u_W1'].T, p['u_b1'].reshape(1, 2 * HIDDEN),
      p['u_W2'].T, p['u_b2'].reshape(1, HIDDEN),
      p['ln_g'].reshape(1, HIDDEN), p['ln_b'].reshape(1, HIDDEN))


def _readout_body(f_ref, an_ref, w1t_ref, b1_ref, w2t_ref, b2_ref,
                  w3t_ref, b3_ref, ae_ref, out_ref):
    f = f_ref[...]
    e = _silu(jnp.dot(f, w1t_ref[...], preferred_element_type=jnp.float32) + b1_ref[...])
    e = _silu(jnp.dot(e, w2t_ref[...], preferred_element_type=jnp.float32) + b2_ref[...])
    e = jnp.dot(e, w3t_ref[...], preferred_element_type=jnp.float32) + b3_ref[...]
    an = an_ref[...]
    ids = jax.lax.broadcasted_iota(jnp.int32, (NBLK, NUM_ATOMS), 1)
    onehot = (an == ids).astype(jnp.float32)
    e = e + jnp.dot(onehot, ae_ref[...], preferred_element_type=jnp.float32)
    s = jnp.sum(e).reshape(1, 1)

    @pl.when(pl.program_id(0) == 0)
    def _():
        out_ref[...] = jnp.zeros((1, 1), jnp.float32)

    out_ref[...] += s


def _readout(feats, atomic_numbers, ro, atomic_e):
    full = lambda shp: pl.BlockSpec(shp, lambda i: tuple(0 for _ in shp))
    out = pl.pallas_call(
        _readout_body,
        grid=(NGRID,),
        in_specs=[
            pl.BlockSpec((NBLK, HIDDEN), lambda i: (i, 0)),
            pl.BlockSpec((NBLK, 1), lambda i: (i, 0)),
            full((HIDDEN, HIDDEN)), full((1, HIDDEN)),
            full((HIDDEN, HIDDEN // 2)), full((1, HIDDEN // 2)),
            full((HIDDEN // 2, 1)), full((1, 1)),
            full((NUM_ATOMS, 1)),
        ],
        out_specs=pl.BlockSpec((1, 1), lambda i: (0, 0)),
        out_shape=jax.ShapeDtypeStruct((1, 1), jnp.float32),
    )(feats, atomic_numbers.reshape(N, 1),
      ro['W1'].T, ro['b1'].reshape(1, HIDDEN),
      ro['W2'].T, ro['b2'].reshape(1, HIDDEN // 2),
      ro['W3'].T, ro['b3'].reshape(1, 1),
      atomic_e)
    return out[0, 0]


def kernel(atomic_numbers, pos, edge_index, centers, widths, node_emb,
           layers, readout, atomic_e):
    row = edge_index[0]
    col = edge_index[1]

    edge_vec = pos[col] - pos[row]
    d2 = jnp.sum(edge_vec * edge_vec, axis=-1)
    ln = jnp.sqrt(d2)
    lutidx = jnp.minimum(
        (jnp.minimum(ln, CUTOFF) * (T / CUTOFF) + 0.5).astype(jnp.int32), T)

    w1t = jnp.stack([p['rn_W1'].T for p in layers])
    b1 = jnp.stack([p['rn_b1'].reshape(1, HIDDEN) for p in layers])
    w2t = jnp.stack([p['rn_W2'].T for p in layers])
    b2 = jnp.stack([p['rn_b2'].reshape(1, HIDDEN) for p in layers])
    luts = _build_luts(centers, widths, w1t, b1, w2t, b2)

    feats = _embed(atomic_numbers, node_emb)

    for li, p in enumerate(layers):
        w = luts[li][lutidx]
        messages = feats[col] * w
        agg = jnp.zeros_like(feats).at[row].add(messages)
        feats = _node_update(feats, agg, p)

    return _readout(feats, atomic_numbers, readout, atomic_e)
